# baseline (speedup 1.0000x reference)
import functools

import jax
import jax.numpy as jnp
from jax import lax
from jax.experimental import pallas as pl
from jax.experimental.pallas import tpu as pltpu
from jax.experimental.pallas import tpu_sc as plsc

NTOKEN = 100000
INPUT_DIM = 128
OUTPUT_DIM = 8
B = 4096
L = 50

NTOKEN_PAD = 100352

_ROWS_PER_BLK = 6272
_N_BLKS = NTOKEN_PAD // _ROWS_PER_BLK
_PACK = 128 // OUTPUT_DIM


def _fold_body(m_ref, w_ref, b_ref, o_ref):
    r = (
        jax.lax.dot_general(
            m_ref[...],
            w_ref[...],
            (((1,), (1,)), ((), ())),
            preferred_element_type=jnp.float32,
        )
        + b_ref[...]
    )
    r3 = r.reshape(_ROWS_PER_BLK // _PACK, _PACK, OUTPUT_DIM)
    o_ref[...] = jnp.concatenate(
        [r3[:, a, :] for a in range(_PACK)], axis=1
    )


def _fold_table(m, w1, b1row):
    return pl.pallas_call(
        _fold_body,
        grid=(_N_BLKS,),
        in_specs=[
            pl.BlockSpec((_ROWS_PER_BLK, INPUT_DIM), lambda i: (i, 0)),
            pl.BlockSpec((OUTPUT_DIM, INPUT_DIM), lambda i: (0, 0)),
            pl.BlockSpec((1, OUTPUT_DIM), lambda i: (0, 0)),
        ],
        out_specs=pl.BlockSpec((_ROWS_PER_BLK // _PACK, 128), lambda i: (i, 0)),
        out_shape=jax.ShapeDtypeStruct((NTOKEN_PAD // _PACK, 128), jnp.float32),
    )(m, w1, b1row)


_NC, _NS = 2, 16
_NW = _NC * _NS
_ROWS_W = B // _NW

_sc_mesh = plsc.VectorSubcoreMesh(core_axis_name="c", subcore_axis_name="s")


_LANES = 16


@functools.partial(
    pl.kernel,
    mesh=_sc_mesh,
    compiler_params=pltpu.CompilerParams(
        use_tc_tiling_on_sc=False, needs_layout_passes=False
    ),
    out_type=jax.ShapeDtypeStruct((L, _NW, OUTPUT_DIM, _ROWS_W), jnp.float32),
    scratch_types=[
        pltpu.VMEM((_ROWS_W, L), jnp.int32),
        pltpu.VMEM((L, _ROWS_W), jnp.int32),
        pltpu.VMEM((L, _ROWS_W, OUTPUT_DIM), jnp.float32),
        pltpu.VMEM((L, OUTPUT_DIM, _ROWS_W), jnp.float32),
        pltpu.SemaphoreType.DMA,
        pltpu.SemaphoreType.DMA,
    ],
)
def _gather(table_hbm, idx_hbm, out_hbm, idx_v, idx_t, rows_v, out_v, sem, osem):
    wid = lax.axis_index("s") * _NC + lax.axis_index("c")
    base = wid * _ROWS_W
    pltpu.sync_copy(idx_hbm.at[pl.ds(base, _ROWS_W)], idx_v)

    lane = lax.iota(jnp.int32, _LANES)

    @plsc.parallel_loop(0, L)
    def tr_idx(j):
        col = jnp.full((_LANES,), j, dtype=jnp.int32)
        for blk in range(_ROWS_W // _LANES):
            row = lane + blk * _LANES
            v = plsc.load_gather(idx_v, [row, col])
            idx_t[j, pl.ds(blk * _LANES, _LANES)] = v

    def fire(j, carry):
        pltpu.async_copy(table_hbm.at[idx_t.at[j]], rows_v.at[j], sem)
        return carry

    lax.fori_loop(0, L, fire, 0)

    _G = 10
    for g in range(L // _G):
        lo, hi = g * _G, (g + 1) * _G

        def wait_grp(j, carry):
            pltpu.make_async_copy(
                table_hbm.at[idx_t.at[j]], rows_v.at[j], sem
            ).wait()
            return carry

        lax.fori_loop(lo, hi, wait_grp, 0)

        @plsc.parallel_loop(lo, hi)
        def tr_rows(j):
            row_j = jnp.full((_LANES,), j, dtype=jnp.int32)
            for blk in range(_ROWS_W // _LANES):
                il = lane + blk * _LANES
                for k in range(OUTPUT_DIM):
                    col = jnp.full((_LANES,), k, dtype=jnp.int32)
                    v = plsc.load_gather(rows_v, [row_j, il, col])
                    out_v[j, k, pl.ds(blk * _LANES, _LANES)] = v

        pltpu.async_copy(
            out_v.at[pl.ds(lo, _G)], out_hbm.at[pl.ds(lo, _G), wid], osem
        )

    for g in range(L // _G):
        lo = g * _G
        pltpu.make_async_copy(
            out_v.at[pl.ds(lo, _G)], out_hbm.at[pl.ds(lo, _G), wid], osem
        ).wait()


def kernel(input, timestamp, train, user_repost_matrix, W1, b1):
    packed = _fold_table(user_repost_matrix, W1, b1.reshape(1, OUTPUT_DIM))
    table = packed.reshape(NTOKEN_PAD, OUTPUT_DIM)
    out4 = _gather(table, input)
    return out4.transpose(1, 3, 0, 2).reshape(B, L, OUTPUT_DIM)


# speedup vs baseline: 1.1032x; 1.1032x over previous
import functools

import jax
import jax.numpy as jnp
from jax import lax
from jax.experimental import pallas as pl
from jax.experimental.pallas import tpu as pltpu
from jax.experimental.pallas import tpu_sc as plsc

NTOKEN = 100000
INPUT_DIM = 128
OUTPUT_DIM = 8
B = 4096
L = 50

NTOKEN_PAD = 100352

_ROWS_PER_BLK = 12544
_N_BLKS = NTOKEN_PAD // _ROWS_PER_BLK
_PACK = 128 // OUTPUT_DIM


def _fold_body(m_ref, w_ref, b_ref, o_ref):
    r = (
        jax.lax.dot_general(
            m_ref[...],
            w_ref[...],
            (((1,), (1,)), ((), ())),
            preferred_element_type=jnp.float32,
        )
        + b_ref[...]
    )
    r3 = r.reshape(_ROWS_PER_BLK // _PACK, _PACK, OUTPUT_DIM)
    o_ref[...] = jnp.concatenate(
        [r3[:, a, :] for a in range(_PACK)], axis=1
    )


def _fold_table(m, w1, b1row):
    return pl.pallas_call(
        _fold_body,
        grid=(_N_BLKS,),
        in_specs=[
            pl.BlockSpec((_ROWS_PER_BLK, INPUT_DIM), lambda i: (i, 0)),
            pl.BlockSpec((OUTPUT_DIM, INPUT_DIM), lambda i: (0, 0)),
            pl.BlockSpec((1, OUTPUT_DIM), lambda i: (0, 0)),
        ],
        out_specs=pl.BlockSpec((_ROWS_PER_BLK // _PACK, 128), lambda i: (i, 0)),
        out_shape=jax.ShapeDtypeStruct((NTOKEN_PAD // _PACK, 128), jnp.float32),
    )(m, w1, b1row)


_NC, _NS = 2, 16
_NW = _NC * _NS
_ROWS_W = B // _NW

_sc_mesh = plsc.VectorSubcoreMesh(core_axis_name="c", subcore_axis_name="s")


_LANES = 16


@functools.partial(
    pl.kernel,
    mesh=_sc_mesh,
    compiler_params=pltpu.CompilerParams(
        use_tc_tiling_on_sc=False, needs_layout_passes=False
    ),
    out_type=jax.ShapeDtypeStruct((L, _NW, OUTPUT_DIM, _ROWS_W), jnp.float32),
    scratch_types=[
        pltpu.VMEM((_ROWS_W, L), jnp.int32),
        pltpu.VMEM((L, _ROWS_W), jnp.int32),
        pltpu.VMEM((L, _ROWS_W, OUTPUT_DIM), jnp.float32),
        pltpu.VMEM((L, OUTPUT_DIM, _ROWS_W), jnp.float32),
        pltpu.SemaphoreType.DMA,
    ],
)
def _gather(table_hbm, idx_hbm, out_hbm, idx_v, idx_t, rows_v, out_v, sem):
    wid = lax.axis_index("s") * _NC + lax.axis_index("c")
    base = wid * _ROWS_W
    pltpu.sync_copy(idx_hbm.at[pl.ds(base, _ROWS_W)], idx_v)

    lane = lax.iota(jnp.int32, _LANES)

    @plsc.parallel_loop(0, L)
    def tr_idx(j):
        col = jnp.full((_LANES,), j, dtype=jnp.int32)
        for blk in range(_ROWS_W // _LANES):
            row = lane + blk * _LANES
            v = plsc.load_gather(idx_v, [row, col])
            idx_t[j, pl.ds(blk * _LANES, _LANES)] = v

    def fire(j, carry):
        pltpu.async_copy(table_hbm.at[idx_t.at[j]], rows_v.at[j], sem)
        return carry

    lax.fori_loop(0, L, fire, 0)

    def wait_all(j, carry):
        pltpu.make_async_copy(
            table_hbm.at[idx_t.at[j]], rows_v.at[j], sem
        ).wait()
        return carry

    lax.fori_loop(0, L, wait_all, 0)

    @plsc.parallel_loop(0, L)
    def tr_rows(j):
        row_j = jnp.full((_LANES,), j, dtype=jnp.int32)
        for blk in range(_ROWS_W // _LANES):
            il = lane + blk * _LANES
            for k in range(OUTPUT_DIM):
                col = jnp.full((_LANES,), k, dtype=jnp.int32)
                v = plsc.load_gather(rows_v, [row_j, il, col])
                out_v[j, k, pl.ds(blk * _LANES, _LANES)] = v

    pltpu.sync_copy(out_v, out_hbm.at[:, wid])


def kernel(input, timestamp, train, user_repost_matrix, W1, b1):
    packed = _fold_table(user_repost_matrix, W1, b1.reshape(1, OUTPUT_DIM))
    table = packed.reshape(NTOKEN_PAD, OUTPUT_DIM)
    out4 = _gather(table, input)
    return out4.transpose(1, 3, 0, 2).reshape(B, L, OUTPUT_DIM)
